# baseline (speedup 1.0000x reference)
import jax
import jax.numpy as jnp
import numpy as np
from jax.experimental import pallas as pl

_N = 10000
_D = 59
_K = 3
_G = 16


def _bn(h, g, be):
    m = jnp.mean(h, axis=0)
    v = jnp.var(h, axis=0)
    return g * (h - m) * jax.lax.rsqrt(v + 1e-5) + be


def _block(h, p):
    h = h @ p["W"] + p["b"]
    h = jax.nn.relu(h)
    return _bn(h, p["g"], p["be"])


def _resmlp(h, blocks):
    hr = h
    for p in blocks:
        h = _block(h, p)
    return h + hr


def _knn_edges(feat, batch):
    n = feat.shape[0]
    sq = jnp.sum(feat * feat, axis=1)
    d2 = sq[:, None] + sq[None, :] - 2.0 * (feat @ feat.T)
    big = jnp.float32(1e30)
    d2 = jnp.where(batch[:, None] != batch[None, :], big, d2)
    idx_n = jnp.arange(n)
    d2 = d2.at[idx_n, idx_n].set(big)
    _, nbr = jax.lax.top_k(-d2, _K)
    src = nbr.reshape(-1)
    dst = jnp.repeat(idx_n, _K)
    return src, dst


def _edgeconv(x, src, dst, nn_fn):
    n = x.shape[0]
    xi = jnp.take(x, dst, axis=0)
    xj = jnp.take(x, src, axis=0)
    m = nn_fn(jnp.concatenate([xi, xj - xi], axis=1))
    agg = jax.ops.segment_sum(m, dst, num_segments=n)
    return agg / jnp.float32(_K)


def _cls_kernel(pooled_ref, w1_ref, b1_ref, g1_ref, be1_ref,
                w2_ref, b2_ref, g2_ref, be2_ref, out_ref):
    h = jnp.dot(pooled_ref[...], w1_ref[...],
                preferred_element_type=jnp.float32) + b1_ref[...]
    h = jnp.maximum(h, 0.0)
    m = jnp.mean(h, axis=0, keepdims=True)
    v = jnp.mean((h - m) * (h - m), axis=0, keepdims=True)
    h = g1_ref[...] * (h - m) * jax.lax.rsqrt(v + 1e-5) + be1_ref[...]
    h2 = jnp.dot(h, w2_ref[...], preferred_element_type=jnp.float32) + b2_ref[...]
    h2 = jnp.maximum(h2, 0.0)
    m2 = jnp.mean(h2, axis=0, keepdims=True)
    v2 = jnp.mean((h2 - m2) * (h2 - m2), axis=0, keepdims=True)
    h2 = g2_ref[...] * (h2 - m2) * jax.lax.rsqrt(v2 + 1e-5) + be2_ref[...]
    out_ref[...] = jax.nn.sigmoid(h2)


def kernel(x, pos, batch, params):
    src, dst = _knn_edges(pos, batch)
    nn1 = lambda h: _resmlp(_block(h, params["c1_mlp"]), params["c1_res"])
    x1 = _edgeconv(x, src, dst, nn1)
    src, dst = _knn_edges(x1, batch)
    x1 = _edgeconv(x1, src, dst, lambda h: _resmlp(h, params["c2_res"]))
    src, dst = _knn_edges(x1, batch)
    x1 = _edgeconv(x1, src, dst, lambda h: _resmlp(h, params["c3_res"]))
    cnt = jax.ops.segment_sum(jnp.ones((x1.shape[0],), jnp.float32), batch,
                              num_segments=_G)
    pooled = jax.ops.segment_sum(x1, batch, num_segments=_G) / jnp.maximum(cnt, 1.0)[:, None]

    p1, p2 = params["cls1"], params["cls2"]
    w2p = jnp.zeros((512, 128), jnp.float32).at[:, 0].set(p2["W"][:, 0])
    vec = lambda a: jnp.broadcast_to(a.reshape(1, -1), (1, a.shape[0]))
    s1 = lambda a: vec(a)
    s2 = lambda a: jnp.full((1, 128), a[0], jnp.float32)
    out = pl.pallas_call(
        _cls_kernel,
        out_shape=jax.ShapeDtypeStruct((_G, 128), jnp.float32),
    )(pooled, p1["W"], s1(p1["b"]), s1(p1["g"]), s1(p1["be"]),
      w2p, s2(p2["b"]), s2(p2["g"]), s2(p2["be"]))
    return out[:, 0]


# speedup vs baseline: 4.3460x; 4.3460x over previous
import jax
import jax.numpy as jnp
from jax import lax
from jax.experimental import pallas as pl

N = 10000
G = 16
K = 3
NP = 10240
BR = 256
TCOL = 256
NT = NP // TCOL
BEn = 256
NB = NP // BEn
BN3 = 512
E3 = 3 * NP
EPS = 1e-5
BIG = 1e30
IBIG = 2 ** 30
F32 = jnp.float32
I32 = jnp.int32


def _knn_body(feat_ref, ft3_ref, sqc3_ref, sqr_ref, cs_ref, ce_ref, out_ref):
    i = pl.program_id(0)
    fr = feat_ref[...]
    sqr = sqr_ref[...]
    cs = cs_ref[...]
    ce = ce_ref[...]
    rowid = i * BR + lax.broadcasted_iota(I32, (BR, 1), 0)
    t0 = jnp.min(cs) // TCOL
    t1 = (jnp.max(ce) + (TCOL - 1)) // TCOL

    def tile_body(t, carry):
        v0, v1, v2, i0, i1, i2 = carry
        c0 = t * TCOL
        fc = ft3_ref[t]
        dot = lax.dot_general(fr, fc, (((1,), (0,)), ((), ())),
                              preferred_element_type=F32)
        d = (sqr + sqc3_ref[t]) - 2.0 * dot
        colg = c0 + lax.broadcasted_iota(I32, (BR, TCOL), 1)
        valid = (colg >= cs) & (colg < ce) & (colg != rowid)
        d = jnp.where(valid, d, BIG)
        tv, ti = [], []
        for _ in range(3):
            vt = jnp.min(d, axis=1, keepdims=True)
            at = jnp.min(jnp.where(d <= vt, colg, IBIG), axis=1, keepdims=True)
            d = jnp.where(colg == at, BIG, d)
            tv.append(vt)
            ti.append(at)
        vals = [v0, v1, v2] + tv
        idxs = [i0, i1, i2] + ti
        used = [jnp.zeros((BR, 1), jnp.bool_) for _ in range(6)]
        outv, outi = [], []
        for _ in range(3):
            bv = jnp.full((BR, 1), BIG, F32)
            bi = jnp.zeros((BR, 1), I32)
            for k in range(6):
                cond = (~used[k]) & (vals[k] < bv)
                bv = jnp.where(cond, vals[k], bv)
                bi = jnp.where(cond, idxs[k], bi)
            found = jnp.zeros((BR, 1), jnp.bool_)
            for k in range(6):
                is_ch = (~used[k]) & (~found) & (vals[k] == bv)
                used[k] = used[k] | is_ch
                found = found | is_ch
            outv.append(bv)
            outi.append(bi)
        return tuple(outv) + tuple(outi)

    bigv = jnp.full((BR, 1), BIG, F32)
    zi = jnp.zeros((BR, 1), I32)
    v0, v1, v2, i0, i1, i2 = lax.fori_loop(
        t0, t1, tile_body, (bigv, bigv, bigv, zi, zi, zi))
    lane = lax.broadcasted_iota(I32, (BR, 8), 1)
    out_ref[...] = jnp.where(lane == 0, i0,
                             jnp.where(lane == 1, i1,
                                       jnp.where(lane == 2, i2, 0)))


def _knn(feat, cs, ce):
    dk = feat.shape[1]
    sq = jnp.sum(feat * feat, axis=1)
    ft3 = feat.T.reshape(dk, NT, TCOL).transpose(1, 0, 2)
    sqc3 = sq.reshape(NT, TCOL)[:, None, :]
    return pl.pallas_call(
        _knn_body,
        grid=(NP // BR,),
        in_specs=[
            pl.BlockSpec((BR, dk), lambda i: (i, 0)),
            pl.BlockSpec((NT, dk, TCOL), lambda i: (0, 0, 0)),
            pl.BlockSpec((NT, 1, TCOL), lambda i: (0, 0, 0)),
            pl.BlockSpec((BR, 1), lambda i: (i, 0)),
            pl.BlockSpec((BR, 1), lambda i: (i, 0)),
            pl.BlockSpec((BR, 1), lambda i: (i, 0)),
        ],
        out_specs=pl.BlockSpec((BR, 8), lambda i: (i, 0)),
        out_shape=jax.ShapeDtypeStruct((NP, 8), I32),
    )(feat, ft3, sqc3, sq[:, None], cs, ce)


def _edge_layer(a3, w, norm=None, xfirst=None, b=None, xshift=None):
    cin = a3.shape[2]
    cout = w.shape[1]
    has_norm = norm is not None
    shift = xshift is not None

    def body(*refs):
        idx = 2
        a_ref, w_ref = refs[0], refs[1]
        norm_ref = x_ref = x2_ref = b_ref = None
        if has_norm:
            norm_ref = refs[idx]; idx += 1
        elif shift:
            x_ref = refs[idx]
            x2_ref = refs[idx + 1]
            b_ref = refs[idx + 2]
            idx += 3
        else:
            x_ref = refs[idx]
            b_ref = refs[idx + 1]
            idx += 2
        h_ref = refs[idx]
        a = a_ref[0]
        if has_norm:
            nr = norm_ref[...]
            a = nr[2:3] * (a - nr[0:1]) * nr[1:2] + nr[3:4]
            h = jnp.dot(a, w_ref[...], preferred_element_type=F32) + nr[4:5]
        elif shift:
            cat = x_ref[...] + (a - x2_ref[...])
            h = jnp.dot(cat, w_ref[...], preferred_element_type=F32) + b_ref[...]
        else:
            xi = x_ref[...]
            cat = jnp.concatenate([xi, a - xi], axis=1)
            h = jnp.dot(cat, w_ref[...], preferred_element_type=F32) + b_ref[...]
        h_ref[0] = jnp.maximum(h, 0.0)

    in_specs = [
        pl.BlockSpec((1, BEn, cin), lambda k, i: (k, i, 0)),
        pl.BlockSpec(w.shape, lambda k, i: (0, 0)),
    ]
    args = [a3, w]
    if has_norm:
        in_specs.append(pl.BlockSpec((8, cin), lambda k, i: (0, 0)))
        args.append(norm)
    else:
        in_specs.append(pl.BlockSpec((BEn, cin), lambda k, i: (i, 0)))
        args.append(xfirst)
        if shift:
            in_specs.append(pl.BlockSpec((BEn, cin), lambda k, i: (i, 0)))
            args.append(xshift)
        in_specs.append(pl.BlockSpec((1, cout), lambda k, i: (0, 0)))
        args.append(b.reshape(1, cout))
    return pl.pallas_call(
        body,
        grid=(3, NB),
        in_specs=in_specs,
        out_specs=pl.BlockSpec((1, BEn, cout), lambda k, i: (k, i, 0)),
        out_shape=jax.ShapeDtypeStruct((3, NP, cout), F32),
    )(*args)


def _mk_norm(h3, g, be, b=None):
    hdo = h3[:, :N, :].transpose(1, 0, 2).reshape(3 * N, h3.shape[2])
    m = jnp.mean(hdo, axis=0)
    v = jnp.var(hdo, axis=0)
    r = lax.rsqrt(v + EPS)
    z = jnp.zeros_like(g)
    rows = [m, r, g, be, b if b is not None else z, z, z, z]
    return jnp.stack(rows, axis=0)


def _bn_rows(nr, h):
    return nr[2:3] * (h - nr[0:1]) * nr[1:2] + nr[3:4]


def _agg1_body(h2_ref, h0_ref, n2_ref, n0_ref, out_ref):
    n2 = n2_ref[...]
    n0 = n0_ref[...]
    s = None
    for k in range(3):
        mk = _bn_rows(n2, h2_ref[k]) + _bn_rows(n0, h0_ref[k])
        s = mk if s is None else s + mk
    out_ref[...] = s


def _agg1(h2, n2, h0, n0, c):
    return pl.pallas_call(
        _agg1_body,
        grid=(NB,),
        in_specs=[
            pl.BlockSpec((3, BEn, c), lambda i: (0, i, 0)),
            pl.BlockSpec((3, BEn, c), lambda i: (0, i, 0)),
            pl.BlockSpec((8, c), lambda i: (0, 0)),
            pl.BlockSpec((8, c), lambda i: (0, 0)),
        ],
        out_specs=pl.BlockSpec((BEn, c), lambda i: (i, 0)),
        out_shape=jax.ShapeDtypeStruct((NP, c), F32),
    )(h2, h0, n2, n0)


def _agg23_body(h_ref, n_ref, xj_ref, x_ref, out_ref):
    n = n_ref[...]
    xb = x_ref[...]
    s = None
    for k in range(3):
        mk = _bn_rows(n, h_ref[k]) + jnp.concatenate(
            [xb, xj_ref[k] - xb], axis=1)
        s = mk if s is None else s + mk
    out_ref[...] = s


def _agg23(h, n, xj3, x, cin):
    c = 2 * cin
    return pl.pallas_call(
        _agg23_body,
        grid=(NB,),
        in_specs=[
            pl.BlockSpec((3, BEn, c), lambda i: (0, i, 0)),
            pl.BlockSpec((8, c), lambda i: (0, 0)),
            pl.BlockSpec((3, BEn, cin), lambda i: (0, i, 0)),
            pl.BlockSpec((BEn, cin), lambda i: (i, 0)),
        ],
        out_specs=pl.BlockSpec((BEn, c), lambda i: (i, 0)),
        out_shape=jax.ShapeDtypeStruct((NP, c), F32),
    )(h, n, xj3, x)


def _pool_body(x_ref, b_ref, ps_ref):
    i = pl.program_id(0)
    bm = b_ref[0]
    oh = (lax.broadcasted_iota(I32, (G, BN3), 0) == bm).astype(F32)

    @pl.when(i == 0)
    def _():
        ps_ref[...] = jnp.zeros_like(ps_ref)

    ps_ref[...] += jnp.dot(oh, x_ref[...], preferred_element_type=F32,
                           precision=lax.Precision.HIGHEST)


def _pool(x3, batch3):
    return pl.pallas_call(
        _pool_body,
        grid=(NP // BN3,),
        in_specs=[
            pl.BlockSpec((BN3, 512), lambda i: (i, 0)),
            pl.BlockSpec((1, 1, BN3), lambda i: (i, 0, 0)),
        ],
        out_specs=pl.BlockSpec((G, 512), lambda i: (0, 0)),
        out_shape=jax.ShapeDtypeStruct((G, 512), F32),
    )(x3, batch3)


def _cls_body(ps_ref, cnt_ref, w1_ref, b1_ref, g1_ref, be1_ref,
              w2_ref, b2_ref, g2_ref, be2_ref, out_ref):
    r = 1.0 / jnp.maximum(cnt_ref[:, 0:1], 1.0)
    pooled = ps_ref[...] * r
    h = jnp.dot(pooled, w1_ref[...], preferred_element_type=F32) + b1_ref[...]
    h = jnp.maximum(h, 0.0)
    m = jnp.mean(h, axis=0, keepdims=True)
    v = jnp.mean((h - m) * (h - m), axis=0, keepdims=True)
    h = g1_ref[...] * (h - m) * lax.rsqrt(v + EPS) + be1_ref[...]
    h2 = jnp.dot(h, w2_ref[...], preferred_element_type=F32) + b2_ref[...]
    h2 = jnp.maximum(h2, 0.0)
    m2 = jnp.mean(h2, axis=0, keepdims=True)
    v2 = jnp.mean((h2 - m2) * (h2 - m2), axis=0, keepdims=True)
    h2 = g2_ref[...] * (h2 - m2) * lax.rsqrt(v2 + EPS) + be2_ref[...]
    out_ref[...] = jax.nn.sigmoid(h2)


def _cls(ps, cntb, p1, p2):
    w2p = jnp.zeros((512, 128), F32).at[:, 0].set(p2["W"][:, 0])
    row = lambda a: a.reshape(1, -1)
    sc = lambda a: jnp.full((1, 128), a[0], F32)
    return pl.pallas_call(
        _cls_body,
        out_shape=jax.ShapeDtypeStruct((G, 128), F32),
    )(ps, cntb, p1["W"], row(p1["b"]), row(p1["g"]), row(p1["be"]),
      w2p, sc(p2["b"]), sc(p2["g"]), sc(p2["be"]))


def _gather_rows(table, idx):
    return jnp.take(table, idx, axis=0)


def _conv1(x, nbr, params):
    src = nbr[:, :K].T.reshape(E3)
    p0 = params["c1_mlp"]
    w0 = p0["W"]
    w0full = jnp.zeros((128, 128), F32).at[:118].set(w0)
    xadd = jnp.zeros((NP, 128), F32).at[:N, :59].set(x)
    xsub = jnp.zeros((NP, 128), F32).at[:N, 59:118].set(x)
    xj = _gather_rows(xsub, src).reshape(3, NP, 128)
    h0 = _edge_layer(xj, w0full, xfirst=xadd, b=p0["b"], xshift=xsub)
    ra, rb = params["c1_res"]
    n0 = _mk_norm(h0, p0["g"], p0["be"], ra["b"])
    h1 = _edge_layer(h0, ra["W"], norm=n0)
    n1 = _mk_norm(h1, ra["g"], ra["be"], rb["b"])
    h2 = _edge_layer(h1, rb["W"], norm=n1)
    n2 = _mk_norm(h2, rb["g"], rb["be"])
    return _agg1(h2, n2, h0, n0, 128) / jnp.float32(K)


def _conv23(x, nbr, blocks, cin):
    src = nbr[:, :K].T.reshape(E3)
    xj3 = _gather_rows(x, src).reshape(3, NP, cin)
    b0, b1, b2 = blocks
    h0 = _edge_layer(xj3, b0["W"], xfirst=x, b=b0["b"])
    n0 = _mk_norm(h0, b0["g"], b0["be"], b1["b"])
    h1 = _edge_layer(h0, b1["W"], norm=n0)
    n1 = _mk_norm(h1, b1["g"], b1["be"], b2["b"])
    h2 = _edge_layer(h1, b2["W"], norm=n1)
    n2 = _mk_norm(h2, b2["g"], b2["be"])
    return _agg23(h2, n2, xj3, x, cin) / jnp.float32(K)


def kernel(x, pos, batch, params):
    batch = batch.astype(I32)
    bounds = jnp.searchsorted(batch, jnp.arange(G + 1, dtype=I32)).astype(I32)
    pad_i = jnp.full((NP - N,), N, I32)
    cs = jnp.concatenate([bounds[batch], pad_i])[:, None]
    ce = jnp.concatenate([bounds[batch + 1], jnp.zeros((NP - N,), I32)])[:, None]

    posp = jnp.zeros((NP, 8), F32).at[:N, :3].set(pos)

    nbr = _knn(posp, cs, ce)
    x1 = _conv1(x, nbr, params)
    nbr = _knn(x1, cs, ce)
    x2 = _conv23(x1, nbr, params["c2_res"], 128)
    nbr = _knn(x2, cs, ce)
    x3 = _conv23(x2, nbr, params["c3_res"], 256)

    bp = jnp.concatenate([batch, jnp.full((NP - N,), G, I32)])
    batch3 = bp.reshape(NP // BN3, 1, BN3)
    ps = _pool(x3, batch3)
    cnt = (bounds[1:] - bounds[:-1]).astype(F32)
    cntb = jnp.broadcast_to(cnt[:, None], (G, 8))
    out = _cls(ps, cntb, params["cls1"], params["cls2"])
    return out[:, 0]
